# Initial kernel scaffold; baseline (speedup 1.0000x reference)
import jax
import jax.numpy as jnp
from jax.experimental import pallas as pl


def kernel(x, r_double, f_double, neighbors, neighbor_mask, Win, W1, b1, W2, b2, Wout, bout):
    raise NotImplementedError("write your pallas kernel here")


# speedup vs baseline: 4410.2254x; 4410.2254x over previous
import functools

import jax
import jax.numpy as jnp
from jax import lax
from jax.experimental import pallas as pl
from jax.experimental.pallas import tpu as pltpu
import jax.experimental.pallas.tpu_sc as plsc

B, At, Nbr, NIN, NF, NOUT, NG = 8, 1024, 32, 128, 128, 128, 64
TOT = B * At * Nbr
NC, NS = 2, 16
NW = NC * NS
CH = 128
RPW = TOT // NW
NCH = RPW // CH
NBUF = 4


def _in2f_body(x_ref, w_ref, o_ref):
    o_ref[...] = jnp.dot(x_ref[...], w_ref[...],
                         preferred_element_type=jnp.float32)


def _in2f(x2d, win):
    return pl.pallas_call(
        _in2f_body,
        grid=(8,),
        in_specs=[
            pl.BlockSpec((1024, NIN), lambda i: (i, 0)),
            pl.BlockSpec((NIN, NF), lambda i: (0, 0)),
        ],
        out_specs=pl.BlockSpec((1024, NF), lambda i: (i, 0)),
        out_shape=jax.ShapeDtypeStruct((B * At, NF), jnp.float32),
    )(x2d, win)


def _sc_gather_body(y_hbm, idx_hbm, out_hbm, idx_v, rows_v, gsem):
    wid = lax.axis_index("s") * NC + lax.axis_index("c")
    base = wid * RPW
    pltpu.sync_copy(idx_hbm.at[pl.ds(wid * NCH, NCH)], idx_v)

    def group(g, carry):
        j0 = g * NBUF
        descs = [
            pltpu.async_copy(y_hbm.at[idx_v.at[j0 + b]], rows_v.at[b], gsem)
            for b in range(NBUF)
        ]
        for b in range(NBUF):
            descs[b].wait()
            pltpu.sync_copy(rows_v.at[b],
                            out_hbm.at[pl.ds(base + (j0 + b) * CH, CH)])
        return carry

    lax.fori_loop(0, NCH // NBUF, group, 0)


def _sc_gather(y2d, gidx2d):
    mesh = plsc.VectorSubcoreMesh(core_axis_name="c", subcore_axis_name="s")
    kern = pl.kernel(
        _sc_gather_body,
        out_type=jax.ShapeDtypeStruct((TOT, NF), jnp.float32),
        mesh=mesh,
        scratch_types=[
            pltpu.VMEM((NCH, CH), jnp.int32),
            pltpu.VMEM((NBUF, CH, NF), jnp.float32),
            pltpu.SemaphoreType.DMA,
        ],
    )
    return kern(y2d, gidx2d)


BA = 128
RB = BA * Nbr


def _ssp(v):
    return (jnp.maximum(v, 0.0) + jnp.log1p(jnp.exp(-jnp.abs(v)))
            - 0.6931471805599453)


def _main_body(f_ref, g_ref, m_ref, s_ref, w1_ref, b1_ref, w2_ref, b2_ref,
               wout_ref, bout_ref, o_ref):
    h = jnp.dot(f_ref[...], w1_ref[...],
                preferred_element_type=jnp.float32) + b1_ref[...]
    wd = jnp.dot(_ssp(h), w2_ref[...],
                 preferred_element_type=jnp.float32) + b2_ref[...]
    z = wd * (g_ref[...] * m_ref[...])
    acc = jnp.dot(s_ref[...], z, preferred_element_type=jnp.float32)
    o_ref[...] = jnp.dot(acc, wout_ref[...],
                         preferred_element_type=jnp.float32) + bout_ref[...]


def _main(f2d, g2d, maskcol, w1, b1, w2, b2, wout, bout):
    nblk = (B * At) // BA
    sel = (jnp.arange(BA, dtype=jnp.int32)[:, None]
           == (jnp.arange(RB, dtype=jnp.int32)[None, :] // Nbr)
           ).astype(jnp.float32)
    return pl.pallas_call(
        _main_body,
        grid=(nblk,),
        in_specs=[
            pl.BlockSpec((RB, NG), lambda i: (i, 0)),
            pl.BlockSpec((RB, NF), lambda i: (i, 0)),
            pl.BlockSpec((RB, 1), lambda i: (i, 0)),
            pl.BlockSpec((BA, RB), lambda i: (0, 0)),
            pl.BlockSpec((NG, NF), lambda i: (0, 0)),
            pl.BlockSpec((1, NF), lambda i: (0, 0)),
            pl.BlockSpec((NF, NF), lambda i: (0, 0)),
            pl.BlockSpec((1, NF), lambda i: (0, 0)),
            pl.BlockSpec((NF, NOUT), lambda i: (0, 0)),
            pl.BlockSpec((1, NOUT), lambda i: (0, 0)),
        ],
        out_specs=pl.BlockSpec((BA, NOUT), lambda i: (i, 0)),
        out_shape=jax.ShapeDtypeStruct((B * At, NOUT), jnp.float32),
    )(f2d, g2d, maskcol, sel, w1, b1, w2, b2, wout, bout)


def kernel(x, r_double, f_double, neighbors, neighbor_mask,
           Win, W1, b1, W2, b2, Wout, bout):
    del r_double
    y2d = _in2f(x.reshape(B * At, NIN), Win)
    gidx = (neighbors.reshape(B, At * Nbr)
            + (jnp.arange(B, dtype=jnp.int32) * At)[:, None])
    g2d = _sc_gather(y2d, gidx.reshape(TOT // CH, CH))
    out = _main(
        f_double.reshape(TOT, NG),
        g2d,
        neighbor_mask.reshape(TOT, 1),
        W1, b1.reshape(1, NF), W2, b2.reshape(1, NF),
        Wout, bout.reshape(1, NOUT),
    )
    return out.reshape(B, At, NOUT)
